# baseline (speedup 1.0000x reference)
import functools

import jax
import jax.numpy as jnp
from jax.experimental import pallas as pl
from jax.experimental.pallas import tpu as pltpu

B = 64
HW = 224
OHW = 112
S = OHW * OHW
C_IN = 3
E = 8
K = 2
N_CLASSES = 1000
G_CH = 16
E_CH = 32
KTAPS = 27
KPAD = 32


def _router_body(a_ref, wg_ref, bg_ref, hg_ref):
    a = a_ref[0]
    h = jnp.dot(a, wg_ref[...], preferred_element_type=jnp.float32)
    h = jax.nn.relu(h + bg_ref[...])
    hg_ref[...] = (jnp.sum(h, axis=0, keepdims=True) / S)[None]


def _routing_body(hg_ref, wl_ref, bl_ref, probs_ref, idx_ref, pw_ref, aux_ref):
    hg = hg_ref[...]
    logits = jnp.dot(hg, wl_ref[...], preferred_element_type=jnp.float32)
    logits = logits + bl_ref[...]
    m = jnp.max(logits, axis=1, keepdims=True)
    ex = jnp.exp(logits - m)
    probs = ex / jnp.sum(ex, axis=1, keepdims=True)
    probs_ref[...] = probs
    iota = jax.lax.broadcasted_iota(jnp.int32, (B, E), 1)
    p1 = jnp.max(probs, axis=1, keepdims=True)
    i1 = jnp.min(jnp.where(probs == p1, iota, E), axis=1, keepdims=True)
    masked = jnp.where(iota == i1, -1.0, probs)
    p2 = jnp.max(masked, axis=1, keepdims=True)
    i2 = jnp.min(jnp.where(masked == p2, iota, E), axis=1, keepdims=True)
    idx_ref[...] = jnp.concatenate([i1, i2], axis=1)
    pw_ref[...] = jnp.concatenate([p1, p2], axis=1)
    mp = jnp.mean(probs, axis=0, keepdims=True)
    d = mp - (1.0 / E)
    aux_ref[...] = jnp.mean(d * d, keepdims=True).reshape(1, 1)


def _expert_body(idx_ref, pw_ref, a_ref, w0_ref, w1_ref, bc0_ref, bc1_ref,
                 l0_ref, l1_ref, bl0_ref, bl1_ref, out_ref):
    b = pl.program_id(0)
    a = a_ref[0]
    p0 = pw_ref[b, 0]
    p1 = pw_ref[b, 1]

    h0 = jnp.dot(a, w0_ref[0], preferred_element_type=jnp.float32)
    h0 = jax.nn.relu(h0 + bc0_ref[0])
    m0 = jnp.sum(h0, axis=0, keepdims=True) / S
    o0 = jnp.dot(m0, l0_ref[0], preferred_element_type=jnp.float32)
    o0 = o0 + bl0_ref[0]

    h1 = jnp.dot(a, w1_ref[0], preferred_element_type=jnp.float32)
    h1 = jax.nn.relu(h1 + bc1_ref[0])
    m1 = jnp.sum(h1, axis=0, keepdims=True) / S
    o1 = jnp.dot(m1, l1_ref[0], preferred_element_type=jnp.float32)
    o1 = o1 + bl1_ref[0]

    out_ref[...] = (p0 * o0 + p1 * o1)[None]


def _im2col(x):
    xpad = jnp.pad(x, ((0, 0), (0, 0), (0, 1), (0, 1)))
    cols = []
    for ky in range(3):
        for kx in range(3):
            for c in range(C_IN):
                cols.append(xpad[:, c, ky:ky + HW:2, kx:kx + HW:2])
    a = jnp.stack(cols, axis=-1)
    a = a.reshape(B, S, KTAPS)
    a = jnp.pad(a, ((0, 0), (0, 0), (0, KPAD - KTAPS)))
    return a.astype(jnp.bfloat16)


def _tap_major(w):
    o = w.shape[0]
    wt = w.transpose(2, 3, 1, 0).reshape(KTAPS, o)
    return jnp.pad(wt, ((0, KPAD - KTAPS), (0, 0)))


@jax.jit
def kernel(x, Wg_conv, bg_conv, Wg_lin, bg_lin, We_conv, be_conv, We_lin, be_lin):
    a = _im2col(x)

    wg = _tap_major(Wg_conv).astype(jnp.bfloat16)
    hg = pl.pallas_call(
        _router_body,
        grid=(B,),
        in_specs=[
            pl.BlockSpec((1, S, KPAD), lambda b: (b, 0, 0)),
            pl.BlockSpec((KPAD, G_CH), lambda b: (0, 0)),
            pl.BlockSpec((1, G_CH), lambda b: (0, 0)),
        ],
        out_specs=pl.BlockSpec((1, 1, G_CH), lambda b: (b, 0, 0)),
        out_shape=jax.ShapeDtypeStruct((B, 1, G_CH), jnp.float32),
    )(a, wg, bg_conv.reshape(1, G_CH))
    hg = hg.reshape(B, G_CH)

    probs, idx, pw, aux = pl.pallas_call(
        _routing_body,
        out_shape=(
            jax.ShapeDtypeStruct((B, E), jnp.float32),
            jax.ShapeDtypeStruct((B, K), jnp.int32),
            jax.ShapeDtypeStruct((B, K), jnp.float32),
            jax.ShapeDtypeStruct((1, 1), jnp.float32),
        ),
    )(hg, Wg_lin, bg_lin.reshape(1, E))

    we = jax.vmap(_tap_major)(We_conv).astype(jnp.bfloat16)
    wl = We_lin
    bc = be_conv.reshape(E, 1, E_CH)
    bl = be_lin.reshape(E, 1, N_CLASSES)

    grid_spec = pltpu.PrefetchScalarGridSpec(
        num_scalar_prefetch=2,
        grid=(B,),
        in_specs=[
            pl.BlockSpec((1, S, KPAD), lambda b, idx_r, pw_r: (b, 0, 0)),
            pl.BlockSpec((1, KPAD, E_CH), lambda b, idx_r, pw_r: (idx_r[b, 0], 0, 0)),
            pl.BlockSpec((1, KPAD, E_CH), lambda b, idx_r, pw_r: (idx_r[b, 1], 0, 0)),
            pl.BlockSpec((1, 1, E_CH), lambda b, idx_r, pw_r: (idx_r[b, 0], 0, 0)),
            pl.BlockSpec((1, 1, E_CH), lambda b, idx_r, pw_r: (idx_r[b, 1], 0, 0)),
            pl.BlockSpec((1, E_CH, N_CLASSES), lambda b, idx_r, pw_r: (idx_r[b, 0], 0, 0)),
            pl.BlockSpec((1, E_CH, N_CLASSES), lambda b, idx_r, pw_r: (idx_r[b, 1], 0, 0)),
            pl.BlockSpec((1, 1, N_CLASSES), lambda b, idx_r, pw_r: (idx_r[b, 0], 0, 0)),
            pl.BlockSpec((1, 1, N_CLASSES), lambda b, idx_r, pw_r: (idx_r[b, 1], 0, 0)),
        ],
        out_specs=pl.BlockSpec((1, 1, N_CLASSES), lambda b, idx_r, pw_r: (b, 0, 0)),
    )
    final = pl.pallas_call(
        _expert_body,
        grid_spec=grid_spec,
        out_shape=jax.ShapeDtypeStruct((B, 1, N_CLASSES), jnp.float32),
    )(idx, pw, a, we, we, bc, bc, wl, wl, bl, bl)
    final = final.reshape(B, N_CLASSES)

    return final, probs, aux.reshape(())


# speedup vs baseline: 5.9737x; 5.9737x over previous
import jax
import jax.numpy as jnp
from jax import lax
from jax.experimental import pallas as pl
from jax.experimental.pallas import tpu as pltpu

B = 64
HW = 224
OHW = 112
S = OHW * OHW
C_IN = 3
E = 8
K = 2
N_CLASSES = 1000
G_CH = 16
E_CH = 32
KTAPS = 27


def _router_body(p_ref, wg_ref, bg_ref, hg_ref):
    p = p_ref[0]
    h = jnp.dot(wg_ref[...], p, preferred_element_type=jnp.float32)
    h = jax.nn.relu(h + bg_ref[...])
    hg_ref[...] = (jnp.sum(h, axis=1, keepdims=True) / S)[None]


def _routing_body(hg_ref, wl_ref, bl_ref, probs_ref, idx_ref, pw_ref, aux_ref):
    hg = hg_ref[...]
    logits = jnp.dot(wl_ref[...], hg, preferred_element_type=jnp.float32)
    logits = logits + bl_ref[...]
    m = jnp.max(logits, axis=0, keepdims=True)
    ex = jnp.exp(logits - m)
    probs = ex / jnp.sum(ex, axis=0, keepdims=True)
    probs_ref[...] = probs.T
    iota = lax.broadcasted_iota(jnp.int32, (E, B), 0)
    p1 = jnp.max(probs, axis=0, keepdims=True)
    i1 = jnp.min(jnp.where(probs == p1, iota, E), axis=0, keepdims=True)
    masked = jnp.where(iota == i1, -1.0, probs)
    p2 = jnp.max(masked, axis=0, keepdims=True)
    i2 = jnp.min(jnp.where(masked == p2, iota, E), axis=0, keepdims=True)
    idx_ref[...] = jnp.concatenate([i1, i2], axis=0)
    pw_ref[...] = jnp.concatenate([p1, p2], axis=0)
    mp = jnp.mean(probs, axis=1, keepdims=True)
    d = mp - (1.0 / E)
    aux_ref[...] = jnp.mean(d * d, keepdims=True).reshape(1, 1)


def _expert_body(idx_ref, pw_ref, p_ref, w0_ref, w1_ref, bc0_ref, bc1_ref,
                 l0_ref, l1_ref, bl0_ref, bl1_ref, out_ref):
    b = pl.program_id(0)
    p = p_ref[0]
    p0 = pw_ref[0, b]
    p1 = pw_ref[1, b]

    w = jnp.concatenate([w0_ref[0], w1_ref[0]], axis=0)
    bc = jnp.concatenate([bc0_ref[0], bc1_ref[0]], axis=0)
    h = jnp.dot(w, p, preferred_element_type=jnp.float32)
    h = jax.nn.relu(h + bc)
    mcol = jnp.sum(h, axis=1, keepdims=True) / S
    scale = jnp.concatenate(
        [jnp.full((E_CH, 1), p0, jnp.float32), jnp.full((E_CH, 1), p1, jnp.float32)],
        axis=0)
    mrow = (mcol * scale).reshape(1, 2 * E_CH)
    lcat = jnp.concatenate([l0_ref[0], l1_ref[0]], axis=0)
    o = jnp.dot(mrow, lcat, preferred_element_type=jnp.float32)
    o = o + p0 * bl0_ref[0] + p1 * bl1_ref[0]
    out_ref[...] = o[None]


@jax.jit
def kernel(x, Wg_conv, bg_conv, Wg_lin, bg_lin, We_conv, be_conv, We_lin, be_lin):
    patches = lax.conv_general_dilated_patches(
        x, (3, 3), (2, 2), 'SAME',
        dimension_numbers=('NCHW', 'OIHW', 'NCHW'))
    p = patches.reshape(B, KTAPS, S).astype(jnp.bfloat16)

    wg = Wg_conv.reshape(G_CH, KTAPS).astype(jnp.bfloat16)
    hg = pl.pallas_call(
        _router_body,
        grid=(B,),
        in_specs=[
            pl.BlockSpec((1, KTAPS, S), lambda b: (b, 0, 0)),
            pl.BlockSpec((G_CH, KTAPS), lambda b: (0, 0)),
            pl.BlockSpec((G_CH, 1), lambda b: (0, 0)),
        ],
        out_specs=pl.BlockSpec((1, G_CH, 1), lambda b: (b, 0, 0)),
        out_shape=jax.ShapeDtypeStruct((B, G_CH, 1), jnp.float32),
    )(p, wg, bg_conv.reshape(G_CH, 1))
    hg = hg.reshape(B, G_CH).T

    probs, idx, pw, aux = pl.pallas_call(
        _routing_body,
        out_shape=(
            jax.ShapeDtypeStruct((B, E), jnp.float32),
            jax.ShapeDtypeStruct((K, B), jnp.int32),
            jax.ShapeDtypeStruct((K, B), jnp.float32),
            jax.ShapeDtypeStruct((1, 1), jnp.float32),
        ),
    )(hg, Wg_lin.T, bg_lin.reshape(E, 1))

    we = We_conv.reshape(E, E_CH, KTAPS).astype(jnp.bfloat16)
    wl = We_lin
    bc = be_conv.reshape(E, E_CH, 1)
    bl = be_lin.reshape(E, 1, N_CLASSES)

    grid_spec = pltpu.PrefetchScalarGridSpec(
        num_scalar_prefetch=2,
        grid=(B,),
        in_specs=[
            pl.BlockSpec((1, KTAPS, S), lambda b, idx_r, pw_r: (b, 0, 0)),
            pl.BlockSpec((1, E_CH, KTAPS), lambda b, idx_r, pw_r: (idx_r[0, b], 0, 0)),
            pl.BlockSpec((1, E_CH, KTAPS), lambda b, idx_r, pw_r: (idx_r[1, b], 0, 0)),
            pl.BlockSpec((1, E_CH, 1), lambda b, idx_r, pw_r: (idx_r[0, b], 0, 0)),
            pl.BlockSpec((1, E_CH, 1), lambda b, idx_r, pw_r: (idx_r[1, b], 0, 0)),
            pl.BlockSpec((1, E_CH, N_CLASSES), lambda b, idx_r, pw_r: (idx_r[0, b], 0, 0)),
            pl.BlockSpec((1, E_CH, N_CLASSES), lambda b, idx_r, pw_r: (idx_r[1, b], 0, 0)),
            pl.BlockSpec((1, 1, N_CLASSES), lambda b, idx_r, pw_r: (idx_r[0, b], 0, 0)),
            pl.BlockSpec((1, 1, N_CLASSES), lambda b, idx_r, pw_r: (idx_r[1, b], 0, 0)),
        ],
        out_specs=pl.BlockSpec((1, 1, N_CLASSES), lambda b, idx_r, pw_r: (b, 0, 0)),
    )
    final = pl.pallas_call(
        _expert_body,
        grid_spec=grid_spec,
        out_shape=jax.ShapeDtypeStruct((B, 1, N_CLASSES), jnp.float32),
    )(idx, pw, p, we, we, bc, bc, wl, wl, bl, bl)
    final = final.reshape(B, N_CLASSES)

    return final, probs, aux.reshape(())
